# baseline (speedup 1.0000x reference)
import functools

import numpy as np
import jax
import jax.numpy as jnp
from jax import lax
from jax.experimental import pallas as pl
from jax.experimental.pallas import tpu as pltpu
from jax.experimental.pallas import tpu_sc as plsc

X = 224
Y = 224
B = 4
D = 64
NC = 2
NS = 16
NW = NC * NS
CH = 128


def _build_index_table():
    radius = min(X, Y) / 2
    xc, yc = X / 2, Y / 2
    xs, ys = np.ogrid[:X, :Y]
    mask = np.sqrt((xs - xc) ** 2 + (ys - yc) ** 2) <= radius
    midx = np.nonzero(mask.reshape(-1))[0].astype(np.int64)
    k = midx.shape[0]
    n = B * k
    gidx = (np.arange(B)[:, None] * (X * Y) + midx[None, :]).reshape(-1)
    n_full = n // CH
    per_w = -(-n_full // NW)
    t = NW * per_w
    starts = np.minimum(np.arange(t) * CH, (n_full - 1) * CH)
    tbl = gidx[starts[:, None] + np.arange(CH)[None, :]]
    tbl = tbl.reshape(NW, per_w, CH)
    tail_start = n_full * CH
    tail_n = n - tail_start
    tail = np.full((CH,), gidx[-1], dtype=np.int64)
    tail[:tail_n] = gidx[tail_start:]
    tbl = np.concatenate([tbl, np.broadcast_to(tail, (NW, 1, CH))], axis=1)
    return k, per_w, tail_start, tail_n, tbl.astype(np.int32)


K_ROWS, PER_W, TAIL_START, TAIL_N, _IDX_TBL_NP = _build_index_table()
N_ROWS = B * K_ROWS

_mesh = plsc.VectorSubcoreMesh(
    core_axis_name="c", subcore_axis_name="s", num_cores=NC, num_subcores=NS
)


N_FULL = N_ROWS // CH


@functools.partial(
    pl.kernel,
    out_type=jax.ShapeDtypeStruct((N_ROWS, D), jnp.float32),
    mesh=_mesh,
    scratch_types=[
        pltpu.VMEM((PER_W + 1, CH), jnp.int32),
        pltpu.VMEM((CH, D), jnp.float32),
        pltpu.SemaphoreType.DMA,
    ],
    compiler_params=pltpu.CompilerParams(use_tc_tiling_on_sc=False),
)
def _gather_kernel(flat_hbm, idx_hbm, out_hbm, idx_v, rows_v, gsem):
    wid = lax.axis_index("s") * NC + lax.axis_index("c")
    pltpu.sync_copy(idx_hbm.at[wid], idx_v)

    def step(c, carry):
        o = jnp.minimum((wid * PER_W + c) * CH, (N_FULL - 1) * CH)
        pltpu.async_copy(flat_hbm.at[idx_v.at[c]], rows_v, gsem).wait()
        pltpu.sync_copy(rows_v, out_hbm.at[pl.ds(o, CH)])
        return carry

    lax.fori_loop(0, PER_W, step, 0)

    @pl.when(wid == 0)
    def _tail():
        pltpu.async_copy(flat_hbm.at[idx_v.at[PER_W]], rows_v, gsem).wait()
        pltpu.sync_copy(
            rows_v.at[pl.ds(0, TAIL_N)], out_hbm.at[pl.ds(TAIL_START, TAIL_N)]
        )


def kernel(tensor):
    flat = tensor.reshape(B * X * Y, D)
    idx = jnp.asarray(_IDX_TBL_NP)
    out = _gather_kernel(flat, idx)
    return out.reshape(B, K_ROWS, D)
